# Initial kernel scaffold; baseline (speedup 1.0000x reference)
import functools
import jax
import jax.numpy as jnp
from jax import lax
from jax.experimental import pallas as pl
from jax.experimental.pallas import tpu as pltpu
from jax.experimental.pallas import tpu_sc as plsc

NUM_EDGES = 3_200_000
NUM_BUS = 100_000
ROW = 128
NROWS = NUM_EDGES // ROW
NCORES = 2
NSUB = 16
NW = NCORES * NSUB
ROWS_PER_W = NROWS // NW
TAIL_ROWS = NROWS - ROWS_PER_W * NW
CHUNK = 128
NFULL = ROWS_PER_W // CHUNK
CTAIL = ROWS_PER_W - NFULL * CHUNK
NB_PAD = 100_352
ZSLICE = NB_PAD // NSUB
ZB = 1568


def _sc_body(idx_hbm, p_hbm, q_hbm, pp_hbm, qp_hbm,
             idx_v, pv, qv, zb, acc_p, acc_q):
    c = lax.axis_index("c")
    s = lax.axis_index("s")
    w = c * NSUB + s

    def zstore(i, _):
        zb[pl.ds(i * 16, 16)] = jnp.zeros((16,), jnp.float32)
        return 0
    lax.fori_loop(0, ZB // 16, zstore, 0)
    base = s * ZSLICE
    for k in range(ZSLICE // ZB):
        pltpu.sync_copy(zb, acc_p.at[pl.ds(base + k * ZB, ZB)])
        pltpu.sync_copy(zb, acc_q.at[pl.ds(base + k * ZB, ZB)])
    plsc.subcore_barrier()

    row0 = w * ROWS_PER_W

    def do_rows(r0, n):
        pltpu.sync_copy(idx_hbm.at[0, pl.ds(r0, n)], idx_v.at[pl.ds(0, n)])
        pltpu.sync_copy(p_hbm.at[pl.ds(r0, n)], pv.at[pl.ds(0, n)])
        pltpu.sync_copy(q_hbm.at[pl.ds(r0, n)], qv.at[pl.ds(0, n)])
        pltpu.sync_copy(pv.at[pl.ds(0, n)], acc_p.at[idx_v.at[pl.ds(0, n)]],
                        add=True)
        pltpu.sync_copy(qv.at[pl.ds(0, n)], acc_q.at[idx_v.at[pl.ds(0, n)]],
                        add=True)

    for i in range(NFULL):
        do_rows(row0 + i * CHUNK, CHUNK)
    do_rows(row0 + NFULL * CHUNK, CTAIL)

    @pl.when(w < TAIL_ROWS)
    def _():
        do_rows(NW * ROWS_PER_W + w, 1)

    plsc.subcore_barrier()

    pltpu.sync_copy(acc_p.at[pl.ds(s * ZSLICE, ZSLICE)],
                    pp_hbm.at[c, pl.ds(s * ZSLICE, ZSLICE)])
    pltpu.sync_copy(acc_q.at[pl.ds(s * ZSLICE, ZSLICE)],
                    qp_hbm.at[c, pl.ds(s * ZSLICE, ZSLICE)])


_sc_scatter = functools.partial(
    pl.kernel,
    out_type=(jax.ShapeDtypeStruct((NCORES, NB_PAD), jnp.float32),
              jax.ShapeDtypeStruct((NCORES, NB_PAD), jnp.float32)),
    mesh=plsc.VectorSubcoreMesh(core_axis_name="c", subcore_axis_name="s"),
    scratch_types=[
        pltpu.VMEM((CHUNK, ROW), jnp.int32),
        pltpu.VMEM((CHUNK, ROW), jnp.float32),
        pltpu.VMEM((CHUNK, ROW), jnp.float32),
        pltpu.VMEM((ZB,), jnp.float32),
        pltpu.VMEM_SHARED((NB_PAD,), jnp.float32),
        pltpu.VMEM_SHARED((NB_PAD,), jnp.float32),
    ],
)(_sc_body)


def _combine_body(pp_ref, qp_ref, po_ref, qo_ref):
    po_ref[...] = pp_ref[0, :] + pp_ref[1, :]
    qo_ref[...] = qp_ref[0, :] + qp_ref[1, :]


_CB = 12_544

_combine = pl.pallas_call(
    _combine_body,
    grid=(NB_PAD // _CB,),
    in_specs=[pl.BlockSpec((NCORES, _CB), lambda i: (0, i)),
              pl.BlockSpec((NCORES, _CB), lambda i: (0, i))],
    out_specs=[pl.BlockSpec((_CB,), lambda i: (i,)),
               pl.BlockSpec((_CB,), lambda i: (i,))],
    out_shape=(jax.ShapeDtypeStruct((NB_PAD,), jnp.float32),
               jax.ShapeDtypeStruct((NB_PAD,), jnp.float32)),
)


def kernel(Pft, Qft, edge_index, num_bus):
    idx3 = edge_index.astype(jnp.int32).reshape(NCORES, NROWS, ROW)
    p3 = Pft.reshape(NROWS, ROW)
    q3 = Qft.reshape(NROWS, ROW)
    pp, qp = _sc_scatter(idx3, p3, q3)
    P, Q = _combine(pp, qp)
    return P[:NUM_BUS], Q[:NUM_BUS]


# speedup vs baseline: 45.7891x; 45.7891x over previous
import functools
import jax
import jax.numpy as jnp
from jax import lax
from jax.experimental import pallas as pl
from jax.experimental.pallas import tpu as pltpu
from jax.experimental.pallas import tpu_sc as plsc

NUM_EDGES = 3_200_000
NUM_BUS = 100_000
ROW = 128
NROWS = NUM_EDGES // ROW
NCORES = 2
NSUB = 16
NW = NCORES * NSUB
ROWS_BIG = 784
N_BIG = 21
CHUNK = 128
NFULL = 6
NB_PAD = 100_352
ZSLICE = NB_PAD // NSUB
ZB = 1568


def _sc_body(idx_hbm, p_hbm, q_hbm, pp_hbm, qp_hbm,
             idx_v, pv, qv, zb, acc_p, acc_q, sem):
    c = lax.axis_index("c")
    s = lax.axis_index("s")
    w = c * NSUB + s

    def zstore(i, _):
        zb[pl.ds(i * 16, 16)] = jnp.zeros((16,), jnp.float32)
        return 0
    lax.fori_loop(0, ZB // 16, zstore, 0)
    base = s * ZSLICE
    for k in range(ZSLICE // ZB):
        pltpu.sync_copy(zb, acc_p.at[pl.ds(base + k * ZB, ZB)])
        pltpu.sync_copy(zb, acc_q.at[pl.ds(base + k * ZB, ZB)])
    plsc.subcore_barrier()

    row0 = w * ROWS_BIG - jnp.maximum(w - N_BIG, 0) * 8

    def do_rows(r0, n):
        pltpu.sync_copy(idx_hbm.at[0, pl.ds(r0, n)], idx_v.at[pl.ds(0, n)])
        pltpu.sync_copy(p_hbm.at[pl.ds(r0, n)], pv.at[pl.ds(0, n)])
        pltpu.sync_copy(q_hbm.at[pl.ds(r0, n)], qv.at[pl.ds(0, n)])

        def fire(i, _):
            pltpu.async_copy(pv.at[i], acc_p.at[idx_v.at[i]], sem, add=True)
            pltpu.async_copy(qv.at[i], acc_q.at[idx_v.at[i]], sem, add=True)
            return 0
        lax.fori_loop(0, n, fire, 0)

        def drain(i, _):
            pltpu.make_async_copy(pv.at[i], acc_p.at[idx_v.at[i]], sem).wait()
            pltpu.make_async_copy(qv.at[i], acc_q.at[idx_v.at[i]], sem).wait()
            return 0
        lax.fori_loop(0, n, drain, 0)

    for i in range(NFULL):
        do_rows(row0 + i * CHUNK, CHUNK)
    do_rows(row0 + NFULL * CHUNK, 8)

    @pl.when(w < N_BIG)
    def _():
        do_rows(row0 + NFULL * CHUNK + 8, 8)

    plsc.subcore_barrier()

    pltpu.sync_copy(acc_p.at[pl.ds(s * ZSLICE, ZSLICE)],
                    pp_hbm.at[c, pl.ds(s * ZSLICE, ZSLICE)])
    pltpu.sync_copy(acc_q.at[pl.ds(s * ZSLICE, ZSLICE)],
                    qp_hbm.at[c, pl.ds(s * ZSLICE, ZSLICE)])


_sc_scatter = functools.partial(
    pl.kernel,
    out_type=(jax.ShapeDtypeStruct((NCORES, NB_PAD), jnp.float32),
              jax.ShapeDtypeStruct((NCORES, NB_PAD), jnp.float32)),
    mesh=plsc.VectorSubcoreMesh(core_axis_name="c", subcore_axis_name="s"),
    scratch_types=[
        pltpu.VMEM((CHUNK, ROW), jnp.int32),
        pltpu.VMEM((CHUNK, ROW), jnp.float32),
        pltpu.VMEM((CHUNK, ROW), jnp.float32),
        pltpu.VMEM((ZB,), jnp.float32),
        pltpu.VMEM_SHARED((NB_PAD,), jnp.float32),
        pltpu.VMEM_SHARED((NB_PAD,), jnp.float32),
        pltpu.SemaphoreType.DMA,
    ],
)(_sc_body)


def _combine_body(pp_ref, qp_ref, po_ref, qo_ref):
    po_ref[...] = pp_ref[0, :] + pp_ref[1, :]
    qo_ref[...] = qp_ref[0, :] + qp_ref[1, :]


_CB = 14_336

_combine = pl.pallas_call(
    _combine_body,
    grid=(NB_PAD // _CB,),
    in_specs=[pl.BlockSpec((NCORES, _CB), lambda i: (0, i)),
              pl.BlockSpec((NCORES, _CB), lambda i: (0, i))],
    out_specs=[pl.BlockSpec((_CB,), lambda i: (i,)),
               pl.BlockSpec((_CB,), lambda i: (i,))],
    out_shape=(jax.ShapeDtypeStruct((NB_PAD,), jnp.float32),
               jax.ShapeDtypeStruct((NB_PAD,), jnp.float32)),
)


def kernel(Pft, Qft, edge_index, num_bus):
    idx3 = edge_index.astype(jnp.int32).reshape(NCORES, NROWS, ROW)
    p3 = Pft.reshape(NROWS, ROW)
    q3 = Qft.reshape(NROWS, ROW)
    pp, qp = _sc_scatter(idx3, p3, q3)
    P, Q = _combine(pp, qp)
    return P[:NUM_BUS], Q[:NUM_BUS]
